# baseline (speedup 1.0000x reference)
import functools

import jax
import jax.numpy as jnp
from jax import lax
from jax.experimental import pallas as pl
from jax.experimental.pallas import tpu as pltpu
from jax.experimental.pallas import tpu_sc as plsc

_N = 65536
_C_IN = 144 * 3
_C_OUT = 118 * 3
_LANES = 16
_NC, _NS = 2, 16
_NW = _NC * _NS
_ROWS_PER_W = _N // _NW
_BB = 64
_NBLK = _ROWS_PER_W // _BB
_IDX_N = _BB * _C_OUT
_NGRP = _IDX_N // _LANES

_mesh = plsc.VectorSubcoreMesh(core_axis_name="c", subcore_axis_name="s")


@functools.partial(
    pl.kernel,
    out_type=jax.ShapeDtypeStruct((_N * _C_OUT,), jnp.float32),
    mesh=_mesh,
    scratch_types=[
        pltpu.VMEM((_IDX_N,), jnp.int32),
        pltpu.VMEM((_BB * _C_IN,), jnp.float32),
        pltpu.VMEM((_BB * _C_IN,), jnp.float32),
        pltpu.VMEM((_BB * _C_OUT,), jnp.float32),
        pltpu.VMEM((_BB * _C_OUT,), jnp.float32),
        pltpu.SemaphoreType.DMA,
        pltpu.SemaphoreType.DMA,
        pltpu.SemaphoreType.DMA,
        pltpu.SemaphoreType.DMA,
    ],
    compiler_params=pltpu.CompilerParams(use_tc_tiling_on_sc=False,
                                         needs_layout_passes=False),
)
def _sc_gather(x_hbm, idx_hbm, out_hbm, idx_v,
               in0, in1, ou0, ou1, is0, is1, os0, os1):
    wid = lax.axis_index("s") * _NC + lax.axis_index("c")
    row0 = wid * _ROWS_PER_W
    pltpu.sync_copy(idx_hbm, idx_v)
    ins, ous, isems, osems = (in0, in1), (ou0, ou1), (is0, is1), (os0, os1)

    def in_copy(blk, s):
        off = (row0 + blk * _BB) * _C_IN
        return pltpu.make_async_copy(
            x_hbm.at[pl.ds(off, _BB * _C_IN)], ins[s], isems[s])

    def out_copy(blk, s):
        off = (row0 + blk * _BB) * _C_OUT
        return pltpu.make_async_copy(
            ous[s], out_hbm.at[pl.ds(off, _BB * _C_OUT)], osems[s])

    in_copy(0, 0).start()
    in_copy(1, 1).start()

    def pair_body(p, carry):
        for s in (0, 1):
            blk = p * 2 + s
            in_copy(blk, s).wait()

            @pl.when(p > 0)
            def _():
                out_copy(blk - 2, s).wait()

            in_v, out_v = ins[s], ous[s]

            @plsc.parallel_loop(0, _NGRP, unroll=8)
            def _(g):
                o = g * _LANES
                cv = idx_v[pl.ds(o, _LANES)]
                out_v[pl.ds(o, _LANES)] = plsc.load_gather(in_v, [cv])

            out_copy(blk, s).start()

            @pl.when(blk + 2 < _NBLK)
            def _():
                in_copy(blk + 2, s).start()
        return carry

    lax.fori_loop(0, _NBLK // 2, pair_body, 0)
    out_copy(_NBLK - 2, 0).wait()
    out_copy(_NBLK - 1, 1).wait()


@jax.jit
def kernel(joints, joint_maps):
    x = joints.reshape(_N * _C_IN)
    jm3 = joint_maps.astype(jnp.int32) * 3
    src = jnp.arange(_C_OUT, dtype=jnp.int32)
    cols = jm3[src // 3] + src % 3
    idx = (jnp.arange(_BB, dtype=jnp.int32)[:, None] * _C_IN
           + cols[None, :]).reshape(_IDX_N)
    out = _sc_gather(x, idx)
    return out.reshape(_N, _C_OUT // 3, 3)


# speedup vs baseline: 31.0602x; 31.0602x over previous
import functools

import jax
import jax.numpy as jnp
from jax import lax
from jax.experimental import pallas as pl
from jax.experimental.pallas import tpu as pltpu
from jax.experimental.pallas import tpu_sc as plsc

_N = 65536
_C_IN = 144 * 3
_C_OUT = 118 * 3
_LANES = 16
_NC, _NS = 2, 16
_NW = _NC * _NS
_ROWS_PER_W = _N // _NW
_BB = 64
_NBLK = _ROWS_PER_W // _BB
_GRP_OFF = tuple(min(g * _LANES, _C_OUT - _LANES)
                 for g in range((_C_OUT + _LANES - 1) // _LANES))
_NGRP = len(_GRP_OFF)

_mesh = plsc.VectorSubcoreMesh(core_axis_name="c", subcore_axis_name="s")


@functools.partial(
    pl.kernel,
    out_type=jax.ShapeDtypeStruct((_N, _C_OUT), jnp.float32),
    mesh=_mesh,
    scratch_types=[
        pltpu.VMEM((_NGRP * _LANES,), jnp.int32),
        pltpu.VMEM((_BB, _C_IN), jnp.float32),
        pltpu.VMEM((_BB, _C_IN), jnp.float32),
        pltpu.VMEM((_BB, _C_OUT), jnp.float32),
        pltpu.VMEM((_BB, _C_OUT), jnp.float32),
        pltpu.SemaphoreType.DMA,
        pltpu.SemaphoreType.DMA,
        pltpu.SemaphoreType.DMA,
        pltpu.SemaphoreType.DMA,
    ],
    compiler_params=pltpu.CompilerParams(use_tc_tiling_on_sc=True,
                                         needs_layout_passes=False),
)
def _sc_gather(x_hbm, cols_hbm, out_hbm, cols_v,
               in0, in1, ou0, ou1, is0, is1, os0, os1):
    wid = lax.axis_index("s") * _NC + lax.axis_index("c")
    row0 = wid * _ROWS_PER_W
    pltpu.sync_copy(cols_hbm, cols_v)
    cvs = [cols_v[pl.ds(g * _LANES, _LANES)] for g in range(_NGRP)]
    ins, ous, isems, osems = (in0, in1), (ou0, ou1), (is0, is1), (os0, os1)

    def in_copy(blk, s):
        return pltpu.make_async_copy(
            x_hbm.at[pl.ds(row0 + blk * _BB, _BB), :], ins[s], isems[s])

    def out_copy(blk, s):
        return pltpu.make_async_copy(
            ous[s], out_hbm.at[pl.ds(row0 + blk * _BB, _BB), :], osems[s])

    in_copy(0, 0).start()
    in_copy(1, 1).start()

    def pair_body(p, carry):
        for s in (0, 1):
            blk = p * 2 + s
            in_copy(blk, s).wait()

            @pl.when(p > 0)
            def _():
                out_copy(blk - 2, s).wait()

            in_v, out_v = ins[s], ous[s]

            @plsc.parallel_loop(0, _BB, unroll=4)
            def _(b):
                bvec = lax.broadcast(b, (_LANES,))
                for og, cv in zip(_GRP_OFF, cvs):
                    out_v[b, pl.ds(og, _LANES)] = plsc.load_gather(
                        in_v, [bvec, cv])

            out_copy(blk, s).start()

            @pl.when(blk + 2 < _NBLK)
            def _():
                in_copy(blk + 2, s).start()
        return carry

    lax.fori_loop(0, _NBLK // 2, pair_body, 0)
    out_copy(_NBLK - 2, 0).wait()
    out_copy(_NBLK - 1, 1).wait()


@jax.jit
def kernel(joints, joint_maps):
    x = joints.reshape(_N, _C_IN)
    jm3 = joint_maps.astype(jnp.int32) * 3
    src = jnp.array([og + l for og in _GRP_OFF for l in range(_LANES)],
                    dtype=jnp.int32)
    cols = jm3[src // 3] + src % 3
    out = _sc_gather(x, cols)
    return out.reshape(_N, _C_OUT // 3, 3)


# speedup vs baseline: 410.4644x; 13.2151x over previous
import functools

import jax
import jax.numpy as jnp
from jax import lax
from jax.experimental import pallas as pl
from jax.experimental.pallas import tpu as pltpu
from jax.experimental.pallas import tpu_sc as plsc

_B = 65536
_J_IN = 144
_J_OUT = 118
_NC, _NS = 2, 16
_NW = _NC * _NS
_CHUNK = 16384
_NCHUNK = _B // _CHUNK
_NTASK = 3 * _J_OUT * _NCHUNK
_STEPS = -(-_NTASK // _NW)
_NBUF = 4

_mesh = plsc.VectorSubcoreMesh(core_axis_name="c", subcore_axis_name="s")


@functools.partial(
    pl.kernel,
    out_type=jax.ShapeDtypeStruct((3, _J_OUT, _B), jnp.float32),
    mesh=_mesh,
    scratch_types=[
        pltpu.VMEM((144,), jnp.int32),
        *[pltpu.VMEM((_CHUNK,), jnp.float32) for _ in range(_NBUF)],
        *[pltpu.SemaphoreType.DMA for _ in range(2 * _NBUF)],
    ],
    compiler_params=pltpu.CompilerParams(use_tc_tiling_on_sc=True,
                                         needs_layout_passes=False),
)
def _sc_copy(x_hbm, jm_hbm, out_hbm, jm_v, *bufs_and_sems):
    bufs = bufs_and_sems[:_NBUF]
    isems = bufs_and_sems[_NBUF:2 * _NBUF]
    osems = bufs_and_sems[2 * _NBUF:3 * _NBUF]
    wid = lax.axis_index("s") * _NC + lax.axis_index("c")
    pltpu.sync_copy(jm_hbm, jm_v)

    def addr(i):
        t = wid + i * _NW
        k = t % _NCHUNK
        p = t // _NCHUNK
        c = p // _J_OUT
        r = p % _J_OUT
        j = jm_v[pl.ds(r, 16)][0]
        return c, r, j, k * _CHUNK

    def in_copy(i):
        c, _, j, off = addr(i)
        return pltpu.make_async_copy(
            x_hbm.at[c, j, pl.ds(off, _CHUNK)], bufs[i % _NBUF],
            isems[i % _NBUF])

    def out_copy(i):
        c, r, _, off = addr(i)
        return pltpu.make_async_copy(
            bufs[i % _NBUF], out_hbm.at[c, r, pl.ds(off, _CHUNK)],
            osems[i % _NBUF])

    def step(i):
        in_copy(i).wait()
        out_copy(i).start()
        nxt = i + _NBUF // 2
        if nxt < _STEPS:
            if nxt - _NBUF >= 0:
                out_copy(nxt - _NBUF).wait()

            def start_next():
                in_copy(nxt).start()
            if (nxt + 1) * _NW <= _NTASK:
                start_next()
            else:
                pl.when(wid < _NTASK - nxt * _NW)(start_next)

    for i in range(_NBUF // 2):
        if (i + 1) * _NW <= _NTASK:
            in_copy(i).start()
        else:
            pl.when(wid < _NTASK - i * _NW)(lambda i=i: in_copy(i).start())

    for i in range(_STEPS):
        if (i + 1) * _NW <= _NTASK:
            step(i)
        else:
            pl.when(wid < _NTASK - i * _NW)(lambda i=i: step(i))

    for i in range(max(_STEPS - _NBUF, 0), _STEPS):
        if (i + 1) * _NW <= _NTASK:
            out_copy(i).wait()
        else:
            pl.when(wid < _NTASK - i * _NW)(lambda i=i: out_copy(i).wait())


@jax.jit
def kernel(joints, joint_maps):
    xt = joints.transpose(2, 1, 0)
    jm = jnp.pad(joint_maps.astype(jnp.int32), (0, 144 - _J_OUT))
    out_t = _sc_copy(xt, jm)
    return out_t.transpose(2, 1, 0)


# speedup vs baseline: 414.5721x; 1.0100x over previous
import functools

import jax
import jax.numpy as jnp
from jax import lax
from jax.experimental import pallas as pl
from jax.experimental.pallas import tpu as pltpu
from jax.experimental.pallas import tpu_sc as plsc

_B = 65536
_J_IN = 144
_J_OUT = 118
_NC, _NS = 2, 16
_NW = _NC * _NS
_CHUNK = 16384
_NCHUNK = _B // _CHUNK
_NTASK = 3 * _J_OUT * _NCHUNK
_STEPS = -(-_NTASK // _NW)
_NBUF = 6

_mesh = plsc.VectorSubcoreMesh(core_axis_name="c", subcore_axis_name="s")


@functools.partial(
    pl.kernel,
    out_type=jax.ShapeDtypeStruct((3, _J_OUT, _B), jnp.float32),
    mesh=_mesh,
    scratch_types=[
        pltpu.VMEM((144,), jnp.int32),
        *[pltpu.VMEM((_CHUNK,), jnp.float32) for _ in range(_NBUF)],
        *[pltpu.SemaphoreType.DMA for _ in range(2 * _NBUF)],
    ],
    compiler_params=pltpu.CompilerParams(use_tc_tiling_on_sc=True,
                                         needs_layout_passes=False),
)
def _sc_copy(x_hbm, jm_hbm, out_hbm, jm_v, *bufs_and_sems):
    bufs = bufs_and_sems[:_NBUF]
    isems = bufs_and_sems[_NBUF:2 * _NBUF]
    osems = bufs_and_sems[2 * _NBUF:3 * _NBUF]
    wid = lax.axis_index("s") * _NC + lax.axis_index("c")
    pltpu.sync_copy(jm_hbm, jm_v)

    def addr(i):
        t = wid + i * _NW
        k = t % _NCHUNK
        p = t // _NCHUNK
        c = p // _J_OUT
        r = p % _J_OUT
        j = jm_v[pl.ds(r, 16)][0]
        return c, r, j, k * _CHUNK

    def in_copy(i):
        c, _, j, off = addr(i)
        return pltpu.make_async_copy(
            x_hbm.at[c, j, pl.ds(off, _CHUNK)], bufs[i % _NBUF],
            isems[i % _NBUF])

    def out_copy(i):
        c, r, _, off = addr(i)
        return pltpu.make_async_copy(
            bufs[i % _NBUF], out_hbm.at[c, r, pl.ds(off, _CHUNK)],
            osems[i % _NBUF])

    def step(i):
        in_copy(i).wait()
        out_copy(i).start()
        nxt = i + _NBUF // 2
        if nxt < _STEPS:
            if nxt - _NBUF >= 0:
                out_copy(nxt - _NBUF).wait()

            def start_next():
                in_copy(nxt).start()
            if (nxt + 1) * _NW <= _NTASK:
                start_next()
            else:
                pl.when(wid < _NTASK - nxt * _NW)(start_next)

    for i in range(_NBUF // 2):
        if (i + 1) * _NW <= _NTASK:
            in_copy(i).start()
        else:
            pl.when(wid < _NTASK - i * _NW)(lambda i=i: in_copy(i).start())

    for i in range(_STEPS):
        if (i + 1) * _NW <= _NTASK:
            step(i)
        else:
            pl.when(wid < _NTASK - i * _NW)(lambda i=i: step(i))

    for i in range(max(_STEPS - _NBUF, 0), _STEPS):
        if (i + 1) * _NW <= _NTASK:
            out_copy(i).wait()
        else:
            pl.when(wid < _NTASK - i * _NW)(lambda i=i: out_copy(i).wait())


@jax.jit
def kernel(joints, joint_maps):
    xt = joints.transpose(2, 1, 0)
    jm = jnp.pad(joint_maps.astype(jnp.int32), (0, 144 - _J_OUT))
    out_t = _sc_copy(xt, jm)
    return out_t.transpose(2, 1, 0)
